# baseline (device time: 231042 ns/iter reference)
import jax
import jax.numpy as jnp
from jax import lax
from jax.experimental import pallas as pl
from jax.experimental.pallas import tpu as pltpu


CHUNKS = [128] + [256] * 15 + [128]
CN_MAX = max(CHUNKS)
OFFS = [sum(CHUNKS[:i]) for i in range(len(CHUNKS))]
N_CHUNKS = len(CHUNKS)
RECV_SLOTS = 4
K_CHUNKS = 4


def kernel(A, B):
    M, Ks = A.shape
    _, N = B.shape
    m_half = M // 2
    kc = Ks // K_CHUNKS
    assert sum(CHUNKS) == N

    def body(
        a_ref,
        b_ref,
        out_ref,
        a_f32,
        a_bf16,
        bd,
        send_buf,
        recv_buf,
        r_buf,
        asem, bsem, sx_send, sx_recv, sy_send, sy_recv, lsem,
    ):
        my_x = lax.axis_index("x")
        my_y = lax.axis_index("y")
        x_peer = (1 - my_x, my_y)
        y_peer = (my_x, 1 - my_y)
        rows_me = my_y * m_half

        def b_fetch(j):
            c = pltpu.make_async_copy(
                b_ref.at[:, pl.ds(OFFS[j], CHUNKS[j])],
                bd.at[j % 2, :, pl.ds(0, CHUNKS[j])],
                bsem.at[j],
            )
            c.start()
            return c

        a_dmas = []
        for k in range(K_CHUNKS):
            c = pltpu.make_async_copy(
                a_ref.at[pl.ds(rows_me, m_half), pl.ds(k * kc, kc)],
                a_f32.at[:, pl.ds(k * kc, kc)],
                asem.at[k],
            )
            c.start()
            a_dmas.append(c)
        b_dmas = [b_fetch(0)]

        barrier = pltpu.get_barrier_semaphore()
        for nbr in (x_peer, y_peer):
            pl.semaphore_signal(
                barrier, inc=1, device_id=nbr, device_id_type=pl.DeviceIdType.MESH
            )
        pl.semaphore_wait(barrier, 2)

        x_rdmas = []
        y_rdmas = []
        l_copies = []

        def process(i):
            t = i % 2
            ci = CHUNKS[i]
            if i >= 2:
                y_rdmas[i - 2].wait_send()
                l_copies[i - 2].wait()
            x_rdmas[i].wait_recv()
            r_buf[t, :, pl.ds(0, ci)] = (
                send_buf[t, :, pl.ds(0, ci)]
                + recv_buf[i % RECV_SLOTS, :, pl.ds(0, ci)]
            )
            lc = pltpu.make_async_copy(
                r_buf.at[t, :, pl.ds(0, ci)],
                out_ref.at[pl.ds(rows_me, m_half), pl.ds(OFFS[i], ci)],
                lsem.at[i],
            )
            lc.start()
            l_copies.append(lc)
            yr = pltpu.make_async_remote_copy(
                src_ref=r_buf.at[t, :, pl.ds(0, ci)],
                dst_ref=out_ref.at[pl.ds(rows_me, m_half), pl.ds(OFFS[i], ci)],
                send_sem=sy_send.at[i],
                recv_sem=sy_recv.at[i],
                device_id=y_peer,
                device_id_type=pl.DeviceIdType.MESH,
            )
            yr.start()
            y_rdmas.append(yr)

        for j in range(N_CHUNKS):
            s = j % 2
            cj = CHUNKS[j]
            if j + 1 < N_CHUNKS:
                b_dmas.append(b_fetch(j + 1))
            b_dmas[j].wait()
            if j >= 2:
                x_rdmas[j - 2].wait_send()
            if j == 0:
                acc = None
                for k in range(K_CHUNKS):
                    a_dmas[k].wait()
                    a_bf16[:, k * kc:(k + 1) * kc] = a_f32[
                        :, k * kc:(k + 1) * kc
                    ].astype(jnp.bfloat16)
                    d = jnp.dot(
                        a_bf16[:, k * kc:(k + 1) * kc],
                        bd[s, k * kc:(k + 1) * kc, 0:cj].astype(jnp.bfloat16),
                        preferred_element_type=jnp.float32,
                    )
                    acc = d if acc is None else acc + d
                send_buf[s, :, pl.ds(0, cj)] = acc.astype(jnp.bfloat16)
            else:
                send_buf[s, :, pl.ds(0, cj)] = jnp.dot(
                    a_bf16[...],
                    bd[s, :, pl.ds(0, cj)].astype(jnp.bfloat16),
                    preferred_element_type=jnp.float32,
                ).astype(jnp.bfloat16)
            xr = pltpu.make_async_remote_copy(
                src_ref=send_buf.at[s, :, pl.ds(0, cj)],
                dst_ref=recv_buf.at[j % RECV_SLOTS, :, pl.ds(0, cj)],
                send_sem=sx_send.at[j],
                recv_sem=sx_recv.at[j],
                device_id=x_peer,
                device_id_type=pl.DeviceIdType.MESH,
            )
            xr.start()
            x_rdmas.append(xr)
            if j >= 1:
                process(j - 1)
        process(N_CHUNKS - 1)

        for j in (N_CHUNKS - 2, N_CHUNKS - 1):
            x_rdmas[j].wait_send()
            y_rdmas[j].wait_send()
            l_copies[j].wait()
        for yr in y_rdmas:
            yr.wait_recv()

    return pl.pallas_call(
        body,
        out_shape=jax.ShapeDtypeStruct((M, N), jnp.bfloat16),
        in_specs=[
            pl.BlockSpec(memory_space=pl.ANY),
            pl.BlockSpec(memory_space=pl.ANY),
        ],
        out_specs=pl.BlockSpec(memory_space=pl.ANY),
        scratch_shapes=[
            pltpu.VMEM((m_half, Ks), jnp.float32),
            pltpu.VMEM((m_half, Ks), jnp.bfloat16),
            pltpu.VMEM((2, Ks, CN_MAX), jnp.float32),
            pltpu.VMEM((2, m_half, CN_MAX), jnp.bfloat16),
            pltpu.VMEM((RECV_SLOTS, m_half, CN_MAX), jnp.bfloat16),
            pltpu.VMEM((2, m_half, CN_MAX), jnp.bfloat16),
            pltpu.SemaphoreType.DMA((K_CHUNKS,)),
            pltpu.SemaphoreType.DMA((N_CHUNKS,)),
            pltpu.SemaphoreType.DMA((N_CHUNKS,)),
            pltpu.SemaphoreType.DMA((N_CHUNKS,)),
            pltpu.SemaphoreType.DMA((N_CHUNKS,)),
            pltpu.SemaphoreType.DMA((N_CHUNKS,)),
            pltpu.SemaphoreType.DMA((N_CHUNKS,)),
        ],
        compiler_params=pltpu.CompilerParams(
            collective_id=0,
            vmem_limit_bytes=56 * 1024 * 1024,
        ),
    )(A, B)


# device time: 228301 ns/iter; 1.0120x vs baseline; 1.0120x over previous
import jax
import jax.numpy as jnp
from jax import lax
from jax.experimental import pallas as pl
from jax.experimental.pallas import tpu as pltpu


CHUNKS = [128] + [256] * 15 + [128]
CN_MAX = max(CHUNKS)
OFFS = [sum(CHUNKS[:i]) for i in range(len(CHUNKS))]
N_CHUNKS = len(CHUNKS)
RECV_SLOTS = 4
K_CHUNKS = 4


def kernel(A, B):
    M, Ks = A.shape
    _, N = B.shape
    m_half = M // 2
    kc = Ks // K_CHUNKS
    assert sum(CHUNKS) == N

    def body(
        a_ref,
        b_ref,
        out_ref,
        a_f32,
        a_bf16,
        bd,
        send_buf,
        recv_buf,
        r_buf,
        asem, bsem, sx_send, sx_recv, sy_send, sy_recv, lsem,
    ):
        my_x = lax.axis_index("x")
        my_y = lax.axis_index("y")
        x_peer = (1 - my_x, my_y)
        y_peer = (my_x, 1 - my_y)
        rows_me = my_y * m_half

        def b_fetch(j):
            c = pltpu.make_async_copy(
                b_ref.at[:, pl.ds(OFFS[j], CHUNKS[j])],
                bd.at[j % 2, :, pl.ds(0, CHUNKS[j])],
                bsem.at[j],
            )
            c.start()
            return c

        b_dmas = [b_fetch(0)]
        a_dmas = []
        for k in range(K_CHUNKS):
            c = pltpu.make_async_copy(
                a_ref.at[pl.ds(rows_me, m_half), pl.ds(k * kc, kc)],
                a_f32.at[:, pl.ds(k * kc, kc)],
                asem.at[k],
            )
            c.start()
            a_dmas.append(c)

        barrier = pltpu.get_barrier_semaphore()
        for nbr in (x_peer, y_peer):
            pl.semaphore_signal(
                barrier, inc=1, device_id=nbr, device_id_type=pl.DeviceIdType.MESH
            )
        pl.semaphore_wait(barrier, 2)

        for k in range(K_CHUNKS):
            a_dmas[k].wait()
            a_bf16[:, k * kc:(k + 1) * kc] = a_f32[:, k * kc:(k + 1) * kc].astype(
                jnp.bfloat16
            )

        x_rdmas = []
        y_rdmas = []
        l_copies = []

        def process(i):
            t = i % 2
            ci = CHUNKS[i]
            if i >= 2:
                y_rdmas[i - 2].wait_send()
                l_copies[i - 2].wait()
            x_rdmas[i].wait_recv()
            r_buf[t, :, pl.ds(0, ci)] = (
                send_buf[t, :, pl.ds(0, ci)]
                + recv_buf[i % RECV_SLOTS, :, pl.ds(0, ci)]
            )
            lc = pltpu.make_async_copy(
                r_buf.at[t, :, pl.ds(0, ci)],
                out_ref.at[pl.ds(rows_me, m_half), pl.ds(OFFS[i], ci)],
                lsem.at[i],
            )
            lc.start()
            l_copies.append(lc)
            yr = pltpu.make_async_remote_copy(
                src_ref=r_buf.at[t, :, pl.ds(0, ci)],
                dst_ref=out_ref.at[pl.ds(rows_me, m_half), pl.ds(OFFS[i], ci)],
                send_sem=sy_send.at[i],
                recv_sem=sy_recv.at[i],
                device_id=y_peer,
                device_id_type=pl.DeviceIdType.MESH,
            )
            yr.start()
            y_rdmas.append(yr)

        for j in range(N_CHUNKS):
            s = j % 2
            cj = CHUNKS[j]
            if j + 1 < N_CHUNKS:
                b_dmas.append(b_fetch(j + 1))
            b_dmas[j].wait()
            if j >= 2:
                x_rdmas[j - 2].wait_send()
            send_buf[s, :, pl.ds(0, cj)] = jnp.dot(
                a_bf16[...],
                bd[s, :, pl.ds(0, cj)].astype(jnp.bfloat16),
                preferred_element_type=jnp.float32,
            ).astype(jnp.bfloat16)
            xr = pltpu.make_async_remote_copy(
                src_ref=send_buf.at[s, :, pl.ds(0, cj)],
                dst_ref=recv_buf.at[j % RECV_SLOTS, :, pl.ds(0, cj)],
                send_sem=sx_send.at[j],
                recv_sem=sx_recv.at[j],
                device_id=x_peer,
                device_id_type=pl.DeviceIdType.MESH,
            )
            xr.start()
            x_rdmas.append(xr)
            if j >= 1:
                process(j - 1)
        process(N_CHUNKS - 1)

        for j in (N_CHUNKS - 2, N_CHUNKS - 1):
            x_rdmas[j].wait_send()
            y_rdmas[j].wait_send()
            l_copies[j].wait()
        for yr in y_rdmas:
            yr.wait_recv()

    return pl.pallas_call(
        body,
        out_shape=jax.ShapeDtypeStruct((M, N), jnp.bfloat16),
        in_specs=[
            pl.BlockSpec(memory_space=pl.ANY),
            pl.BlockSpec(memory_space=pl.ANY),
        ],
        out_specs=pl.BlockSpec(memory_space=pl.ANY),
        scratch_shapes=[
            pltpu.VMEM((m_half, Ks), jnp.float32),
            pltpu.VMEM((m_half, Ks), jnp.bfloat16),
            pltpu.VMEM((2, Ks, CN_MAX), jnp.float32),
            pltpu.VMEM((2, m_half, CN_MAX), jnp.bfloat16),
            pltpu.VMEM((RECV_SLOTS, m_half, CN_MAX), jnp.bfloat16),
            pltpu.VMEM((2, m_half, CN_MAX), jnp.bfloat16),
            pltpu.SemaphoreType.DMA((K_CHUNKS,)),
            pltpu.SemaphoreType.DMA((N_CHUNKS,)),
            pltpu.SemaphoreType.DMA((N_CHUNKS,)),
            pltpu.SemaphoreType.DMA((N_CHUNKS,)),
            pltpu.SemaphoreType.DMA((N_CHUNKS,)),
            pltpu.SemaphoreType.DMA((N_CHUNKS,)),
            pltpu.SemaphoreType.DMA((N_CHUNKS,)),
        ],
        compiler_params=pltpu.CompilerParams(
            collective_id=0,
            vmem_limit_bytes=56 * 1024 * 1024,
        ),
    )(A, B)
